# baseline (speedup 1.0000x reference)
import functools

import jax
import jax.numpy as jnp
from jax.experimental import pallas as pl
from jax.experimental.pallas import tpu as pltpu

_NUM_EMB = 512
_EMB_DIM = 32
_COMMIT = 0.25
_ROWS = 16 * 64 * 64
_BLK = 4096
_GRID = _ROWS // _BLK


def _vq_body(xt_ref, sume_ref, embT2_ref, emb_ref,
             enc_ref, out_ref, hist_ref, sse_ref):
    i = pl.program_id(0)
    xt = xt_ref[0]
    x = jnp.transpose(xt)
    sumx = jnp.sum(x * x, axis=1, keepdims=True)
    mm2 = jnp.dot(x, embT2_ref[...], preferred_element_type=jnp.float32)
    t = sumx + sume_ref[...]
    dist = t - mm2
    iota = jax.lax.broadcasted_iota(jnp.int32, (_BLK, _NUM_EMB), 1)
    m = jnp.min(dist, axis=1, keepdims=True)
    idx = jnp.min(jnp.where(dist == m, iota, _NUM_EMB), axis=1, keepdims=True)
    enc = (iota == idx).astype(jnp.float32)
    enc_ref[...] = enc
    qt = jax.lax.dot_general(emb_ref[...], enc, (((0,), (1,)), ((), ())),
                             preferred_element_type=jnp.float32)
    out_ref[0] = xt + (qt - xt)
    d = qt - xt
    ones_row = jnp.ones((1, _BLK), jnp.float32)
    part_hist = jnp.dot(ones_row, enc,
                        preferred_element_type=jnp.float32)
    part_sse = jnp.sum(d * d)

    @pl.when(i == 0)
    def _init():
        hist_ref[...] = part_hist
        sse_ref[0, 0] = part_sse

    @pl.when(i != 0)
    def _acc():
        hist_ref[...] = hist_ref[...] + part_hist
        sse_ref[0, 0] = sse_ref[0, 0] + part_sse


@functools.partial(jax.jit, static_argnames=())
def kernel(inputs, embedding):
    xt3 = inputs.reshape(16, _EMB_DIM, 64 * 64)
    sume = jnp.sum(embedding ** 2, axis=1).reshape(1, -1)
    embT2 = embedding.T * 2.0

    n_sub = (64 * 64) // _BLK if _BLK <= 64 * 64 else 1
    blk_hw = min(_BLK, 64 * 64)

    enc, out3, hist, sse = pl.pallas_call(
        _vq_body,
        grid=(_GRID,),
        in_specs=[
            pl.BlockSpec((1, _EMB_DIM, blk_hw),
                         lambda i: (i // n_sub, 0, i % n_sub)),
            pl.BlockSpec((1, _NUM_EMB), lambda i: (0, 0)),
            pl.BlockSpec((_EMB_DIM, _NUM_EMB), lambda i: (0, 0)),
            pl.BlockSpec((_NUM_EMB, _EMB_DIM), lambda i: (0, 0)),
        ],
        out_specs=[
            pl.BlockSpec((_BLK, _NUM_EMB), lambda i: (i, 0)),
            pl.BlockSpec((1, _EMB_DIM, blk_hw),
                         lambda i: (i // n_sub, 0, i % n_sub)),
            pl.BlockSpec((1, _NUM_EMB), lambda i: (0, 0)),
            pl.BlockSpec(memory_space=pltpu.SMEM, block_shape=(1, 1),
                         index_map=lambda i: (0, 0)),
        ],
        out_shape=[
            jax.ShapeDtypeStruct((_ROWS, _NUM_EMB), jnp.float32),
            jax.ShapeDtypeStruct((16, _EMB_DIM, 64 * 64), jnp.float32),
            jax.ShapeDtypeStruct((1, _NUM_EMB), jnp.float32),
            jax.ShapeDtypeStruct((1, 1), jnp.float32),
        ],
    )(xt3, sume, embT2, embedding)

    n_el = _ROWS * _EMB_DIM
    mse = sse[0, 0] / n_el
    loss = mse + _COMMIT * mse
    out = out3.reshape(16, _EMB_DIM, 64, 64)
    avg_probs = hist[0] / _ROWS
    perplexity = jnp.exp(-jnp.sum(avg_probs * jnp.log(avg_probs + 1e-10)))
    return (loss, out, perplexity, enc)
